# Initial kernel scaffold; baseline (speedup 1.0000x reference)
import jax
import jax.numpy as jnp
from jax.experimental import pallas as pl


def kernel(x, edge_index, W_z, U_z, W_r, U_r, W_h, U_h, W_o, T_mean, T_var, G_mean, G_var, eps_t, eps_g):
    raise NotImplementedError("write your pallas kernel here")


# speedup vs baseline: 11.0133x; 11.0133x over previous
import functools

import jax
import jax.numpy as jnp
from jax import lax
from jax.experimental import pallas as pl
from jax.experimental.pallas import tpu as pltpu
from jax.experimental.pallas import tpu_sc as plsc

_N = 10000
_E = 320000
_D = 128
_H = 128
_L = 56
_B = 100
_DEPTH = 3

_NC = 2
_NS = 16
_NT = _NC * _NS
_EPT = _E // _NT
_C = 100
_NCH = _EPT // _C
_NP = 10240
_RPT = _NP // _NS

_R = 2000


def _spmm(hn, src3, dst3, zrows):
    mesh = plsc.VectorSubcoreMesh(core_axis_name="c", subcore_axis_name="s")

    @functools.partial(
        pl.kernel,
        mesh=mesh,
        out_type=jax.ShapeDtypeStruct((2, _NP, _H), jnp.float32),
        scratch_types=[
            pltpu.VMEM((_NCH, _C), jnp.int32),
            pltpu.VMEM((_NCH, _C), jnp.int32),
            pltpu.VMEM((_C, _H), jnp.float32),
            pltpu.VMEM_SHARED((_NP, _H), jnp.float32),
            pltpu.SemaphoreType.DMA,
        ],
    )
    def k(hn_hbm, src_hbm, dst_hbm, z_hbm, out_hbm, src_v, dst_v, rows_v,
          acc_sh, sem):
        cid = lax.axis_index("c")
        sid = lax.axis_index("s")
        gid = sid * _NC + cid
        pltpu.sync_copy(src_hbm.at[gid], src_v)
        pltpu.sync_copy(dst_hbm.at[gid], dst_v)
        base = sid * _RPT
        pltpu.sync_copy(z_hbm, acc_sh.at[pl.ds(base, _RPT)])
        plsc.subcore_barrier()

        @pl.loop(0, _NCH)
        def _(j):
            pltpu.async_copy(hn_hbm.at[src_v.at[j]], rows_v, sem).wait()
            pltpu.sync_copy(rows_v, acc_sh.at[dst_v.at[j]], add=True)

        plsc.subcore_barrier()
        pltpu.sync_copy(acc_sh.at[pl.ds(base, _RPT)],
                        out_hbm.at[cid, pl.ds(base, _RPT)])

    return k(hn, src3, dst3, zrows)


def _phase_a_body(x_ref, w_ref, xw_ref, hn_ref):
    xw = jnp.dot(x_ref[...], w_ref[...], preferred_element_type=jnp.float32)
    xw_ref[...] = xw
    hn_ref[...] = jax.nn.sigmoid(xw[:, :_H]) * jnp.tanh(xw[:, 2 * _H:])


def _phase_a(x, w_all):
    return pl.pallas_call(
        _phase_a_body,
        grid=(_N // _R,),
        in_specs=[
            pl.BlockSpec((_R, _D), lambda i: (i, 0)),
            pl.BlockSpec((_D, 3 * _H), lambda i: (0, 0)),
        ],
        out_specs=[
            pl.BlockSpec((_R, 3 * _H), lambda i: (i, 0)),
            pl.BlockSpec((_R, _H), lambda i: (i, 0)),
        ],
        out_shape=[
            jax.ShapeDtypeStruct((_N, 3 * _H), jnp.float32),
            jax.ShapeDtypeStruct((_N, _H), jnp.float32),
        ],
    )(x, w_all)


def _gru_body(xw_ref, hn_ref, p0_ref, p1_ref, uz_ref, ur_ref, uh_ref, out_ref):
    m = p0_ref[0] + p1_ref[0]
    xw = xw_ref[...]
    hn = hn_ref[...]
    z = jax.nn.sigmoid(
        xw[:, :_H] + jnp.dot(m, uz_ref[...], preferred_element_type=jnp.float32))
    r = jax.nn.sigmoid(
        xw[:, _H:2 * _H]
        + jnp.dot(hn, ur_ref[...], preferred_element_type=jnp.float32))
    g = jnp.tanh(
        xw[:, 2 * _H:]
        + jnp.dot(r * m, uh_ref[...], preferred_element_type=jnp.float32))
    out_ref[...] = (1.0 - z) * m + z * g


def _gru(xw, hn, parts, u_z, u_r, u_h):
    nb = _N // _R
    return pl.pallas_call(
        _gru_body,
        grid=(nb,),
        in_specs=[
            pl.BlockSpec((_R, 3 * _H), lambda i: (i, 0)),
            pl.BlockSpec((_R, _H), lambda i: (i, 0)),
            pl.BlockSpec((1, _R, _H), lambda i: (0, i, 0)),
            pl.BlockSpec((1, _R, _H), lambda i: (1, i, 0)),
            pl.BlockSpec((_H, _H), lambda i: (0, 0)),
            pl.BlockSpec((_H, _H), lambda i: (0, 0)),
            pl.BlockSpec((_H, _H), lambda i: (0, 0)),
        ],
        out_specs=pl.BlockSpec((_R, _H), lambda i: (i, 0)),
        out_shape=jax.ShapeDtypeStruct((_N, _H), jnp.float32),
    )(xw, hn, parts, parts, u_z, u_r, u_h)


def _final_body(x_ref, p_ref, wot_ref, wob_ref, tgm_ref, tgv_ref, eps_ref,
                out_ref):
    m = p_ref[0, :_N, :] + p_ref[1, :_N, :]
    nh = jax.nn.relu(
        jnp.dot(x_ref[...], wot_ref[...], preferred_element_type=jnp.float32)
        + jnp.dot(m, wob_ref[...], preferred_element_type=jnp.float32))
    npb = _N // _B
    rows = lax.broadcasted_iota(jnp.int32, (_B, _N), 0)
    cols = lax.broadcasted_iota(jnp.int32, (_B, _N), 1)
    pool = jnp.where(cols // npb == rows, 1.0 / npb, 0.0)
    gv = jnp.dot(pool, nh, preferred_element_type=jnp.float32)
    mean = jnp.dot(gv, tgm_ref[...], preferred_element_type=jnp.float32)
    logv = -jnp.abs(jnp.dot(gv, tgv_ref[...], preferred_element_type=jnp.float32))
    out_ref[...] = mean + jnp.exp(logv * 0.5) * eps_ref[...]


def _final(x, parts, wo_t, wo_b, tgm, tgv, eps):
    return pl.pallas_call(
        _final_body,
        out_shape=jax.ShapeDtypeStruct((_B, _L), jnp.float32),
    )(x, parts, wo_t, wo_b, tgm, tgv, eps)


def kernel(x, edge_index, W_z, U_z, W_r, U_r, W_h, U_h, W_o,
           T_mean, T_var, G_mean, G_var, eps_t, eps_g):
    src3 = edge_index[0].astype(jnp.int32).reshape(_NT, _NCH, _C)
    dst3 = edge_index[1].astype(jnp.int32).reshape(_NT, _NCH, _C)
    zrows = jnp.zeros((_RPT, _H), jnp.float32)
    w_all = jnp.concatenate([W_z, W_r, W_h], axis=1)

    xw, hn = _phase_a(x, w_all)
    for _ in range(_DEPTH - 1):
        parts = _spmm(hn, src3, dst3, zrows)
        hn = _gru(xw, hn, parts, U_z, U_r, U_h)
    parts = _spmm(hn, src3, dst3, zrows)

    wo_t = W_o[:_D]
    wo_b = W_o[_D:]
    tgm = jnp.concatenate([T_mean, G_mean], axis=1)
    tgv = jnp.concatenate([T_var, G_var], axis=1)
    eps = jnp.concatenate([eps_t, eps_g], axis=1)
    return _final(x, parts, wo_t, wo_b, tgm, tgv, eps)


# speedup vs baseline: 16.4081x; 1.4898x over previous
import functools

import jax
import jax.numpy as jnp
from jax import lax
from jax.experimental import pallas as pl
from jax.experimental.pallas import tpu as pltpu
from jax.experimental.pallas import tpu_sc as plsc

_N = 10000
_E = 320000
_D = 128
_H = 128
_L = 56
_B = 100
_DEPTH = 3

_NC = 2
_NS = 16
_NT = _NC * _NS
_EPT = _E // _NT
_C = 80
_NCH = _EPT // _C
_ZT = 10
_ZR = _N // _ZT

_R = 2000


def _spmm(hn, pk3, zrows):
    mesh = plsc.VectorSubcoreMesh(core_axis_name="c", subcore_axis_name="s")

    @functools.partial(
        pl.kernel,
        mesh=mesh,
        out_type=jax.ShapeDtypeStruct((2, _N, _H), jnp.float32),
        scratch_types=[
            pltpu.VMEM((_NCH, _C), jnp.int32),
            pltpu.VMEM((2, _C), jnp.int32),
            pltpu.VMEM((2, _C), jnp.int32),
            pltpu.VMEM((_C, _H), jnp.float32),
            pltpu.VMEM((_C, _H), jnp.float32),
            pltpu.VMEM_SHARED((_N, _H), jnp.float32),
            pltpu.SemaphoreType.DMA,
            pltpu.SemaphoreType.DMA,
            pltpu.SemaphoreType.DMA,
            pltpu.SemaphoreType.DMA,
        ],
    )
    def k(hn_hbm, pk_hbm, z_hbm, out_hbm, pk_v, si_v, di_v,
          r0, r1, acc_sh, g0, g1, s0, s1):
        cid = lax.axis_index("c")
        sid = lax.axis_index("s")
        gid = sid * _NC + cid
        pltpu.sync_copy(pk_hbm.at[gid], pk_v)

        @pl.when(sid < _ZT)
        def _():
            pltpu.sync_copy(z_hbm, acc_sh.at[pl.ds(sid * _ZR, _ZR)])

        plsc.subcore_barrier()

        def unpack(b, j):
            @pl.loop(0, _C // 16)
            def _(u):
                v = pk_v[j, pl.ds(u * 16, 16)]
                si_v[b, pl.ds(u * 16, 16)] = v >> 14
                di_v[b, pl.ds(u * 16, 16)] = v & 16383

        bufs = ((r0, g0, s0), (r1, g1, s1))

        def gather(b, j):
            buf, gsem, _ = bufs[b]
            unpack(b, j)
            pltpu.make_async_copy(hn_hbm.at[si_v.at[b]], buf, gsem).start()

        def wait_gather(b):
            buf, gsem, _ = bufs[b]
            pltpu.make_async_copy(hn_hbm.at[si_v.at[b]], buf, gsem).wait()

        def scatter(b):
            buf, _, ssem = bufs[b]
            pltpu.async_copy(buf, acc_sh.at[di_v.at[b]], ssem,
                             add=True).wait()

        gather(0, 0)
        gather(1, 1)
        npair = (_NCH - 1) // 2

        @pl.loop(0, npair)
        def _(i):
            j = 2 * i
            wait_gather(0)
            scatter(0)
            gather(0, j + 2)
            wait_gather(1)
            scatter(1)

            @pl.when(i < npair - 1)
            def _():
                gather(1, j + 3)

        wait_gather(0)
        scatter(0)

        plsc.subcore_barrier()

        @pl.when(sid < _ZT)
        def _():
            pltpu.sync_copy(acc_sh.at[pl.ds(sid * _ZR, _ZR)],
                            out_hbm.at[cid, pl.ds(sid * _ZR, _ZR)])

    return k(hn, pk3, zrows)


def _phase_a_body(x_ref, w_ref, xw_ref, hn_ref):
    xw = jnp.dot(x_ref[...], w_ref[...], preferred_element_type=jnp.float32)
    xw_ref[...] = xw
    hn_ref[...] = jax.nn.sigmoid(xw[:, :_H]) * jnp.tanh(xw[:, 2 * _H:])


def _phase_a(x, w_all):
    return pl.pallas_call(
        _phase_a_body,
        grid=(_N // _R,),
        in_specs=[
            pl.BlockSpec((_R, _D), lambda i: (i, 0)),
            pl.BlockSpec((_D, 3 * _H), lambda i: (0, 0)),
        ],
        out_specs=[
            pl.BlockSpec((_R, 3 * _H), lambda i: (i, 0)),
            pl.BlockSpec((_R, _H), lambda i: (i, 0)),
        ],
        out_shape=[
            jax.ShapeDtypeStruct((_N, 3 * _H), jnp.float32),
            jax.ShapeDtypeStruct((_N, _H), jnp.float32),
        ],
    )(x, w_all)


def _gru_body(xw_ref, hn_ref, p0_ref, p1_ref, uz_ref, ur_ref, uh_ref, out_ref):
    f32 = jnp.float32
    m = p0_ref[0] + p1_ref[0]
    xw = xw_ref[...]
    hn = hn_ref[...]
    z = jax.nn.sigmoid(
        xw[:, :_H] + jnp.dot(m, uz_ref[...], preferred_element_type=f32))
    r = jax.nn.sigmoid(
        xw[:, _H:2 * _H]
        + jnp.dot(hn, ur_ref[...], preferred_element_type=f32))
    g = jnp.tanh(
        xw[:, 2 * _H:]
        + jnp.dot(r * m, uh_ref[...], preferred_element_type=f32))
    out_ref[...] = (1.0 - z) * m + z * g


def _gru(xw, hn, parts, u_z, u_r, u_h):
    nb = _N // _R
    return pl.pallas_call(
        _gru_body,
        grid=(nb,),
        in_specs=[
            pl.BlockSpec((_R, 3 * _H), lambda i: (i, 0)),
            pl.BlockSpec((_R, _H), lambda i: (i, 0)),
            pl.BlockSpec((1, _R, _H), lambda i: (0, i, 0)),
            pl.BlockSpec((1, _R, _H), lambda i: (1, i, 0)),
            pl.BlockSpec((_H, _H), lambda i: (0, 0)),
            pl.BlockSpec((_H, _H), lambda i: (0, 0)),
            pl.BlockSpec((_H, _H), lambda i: (0, 0)),
        ],
        out_specs=pl.BlockSpec((_R, _H), lambda i: (i, 0)),
        out_shape=jax.ShapeDtypeStruct((_N, _H), jnp.float32),
    )(xw, hn, parts, parts, u_z, u_r, u_h)


def _final_body(x_ref, p_ref, wot_ref, wob_ref, tgm_ref, tgv_ref, eps_ref,
                out_ref):
    f32 = jnp.float32
    m = p_ref[0] + p_ref[1]
    nh = jax.nn.relu(
        jnp.dot(x_ref[...], wot_ref[...], preferred_element_type=f32)
        + jnp.dot(m, wob_ref[...], preferred_element_type=f32))
    npb = _N // _B
    rows = lax.broadcasted_iota(jnp.int32, (_B, _N), 0)
    cols = lax.broadcasted_iota(jnp.int32, (_B, _N), 1)
    pool = jnp.where(cols // npb == rows, 1.0 / npb, 0.0)
    gv = jnp.dot(pool, nh, preferred_element_type=f32)
    mean = jnp.dot(gv, tgm_ref[...], preferred_element_type=f32)
    logv = -jnp.abs(jnp.dot(gv, tgv_ref[...], preferred_element_type=f32))
    out_ref[...] = mean + jnp.exp(logv * 0.5) * eps_ref[...]


def _final(x, parts, wo_t, wo_b, tgm, tgv, eps):
    return pl.pallas_call(
        _final_body,
        out_shape=jax.ShapeDtypeStruct((_B, _L), jnp.float32),
    )(x, parts, wo_t, wo_b, tgm, tgv, eps)


def kernel(x, edge_index, W_z, U_z, W_r, U_r, W_h, U_h, W_o,
           T_mean, T_var, G_mean, G_var, eps_t, eps_g):
    src = edge_index[0].astype(jnp.int32)
    dst = edge_index[1].astype(jnp.int32)
    pk3 = (src * 16384 + dst).reshape(_NT, _NCH, _C)
    zrows = jnp.zeros((_ZR, _H), jnp.float32)
    w_all = jnp.concatenate([W_z, W_r, W_h], axis=1)

    xw, hn = _phase_a(x, w_all)
    for _ in range(_DEPTH - 1):
        parts = _spmm(hn, pk3, zrows)
        hn = _gru(xw, hn, parts, U_z, U_r, U_h)
    parts = _spmm(hn, pk3, zrows)

    wo_t = W_o[:_D]
    wo_b = W_o[_D:]
    tgm = jnp.concatenate([T_mean, G_mean], axis=1)
    tgv = jnp.concatenate([T_var, G_var], axis=1)
    eps = jnp.concatenate([eps_t, eps_g], axis=1)
    return _final(x, parts, wo_t, wo_b, tgm, tgv, eps)


# speedup vs baseline: 18.2627x; 1.1130x over previous
import functools

import jax
import jax.numpy as jnp
from jax import lax
from jax.experimental import pallas as pl
from jax.experimental.pallas import tpu as pltpu
from jax.experimental.pallas import tpu_sc as plsc

_N = 10000
_E = 320000
_D = 128
_H = 128
_L = 56
_B = 100
_DEPTH = 3

_NC = 2
_NS = 16
_NT = _NC * _NS
_EPT = _E // _NT
_C = 80
_NCH = _EPT // _C
_ZT = 10
_ZR = _N // _ZT
_ZB = 200

_R = 2000


def _spmm(hn, pk3, zrows):
    mesh = plsc.VectorSubcoreMesh(core_axis_name="c", subcore_axis_name="s")

    @functools.partial(
        pl.kernel,
        mesh=mesh,
        out_type=jax.ShapeDtypeStruct((2, _N, _H), jnp.float32),
        scratch_types=[
            pltpu.VMEM((_NCH, _C), jnp.int32),
            pltpu.VMEM((3, _C), jnp.int32),
            pltpu.VMEM((3, _C), jnp.int32),
            pltpu.VMEM((_C, _H), jnp.float32),
            pltpu.VMEM((_C, _H), jnp.float32),
            pltpu.VMEM((_C, _H), jnp.float32),
            pltpu.VMEM_SHARED((_N, _H), jnp.float32),
            pltpu.SemaphoreType.DMA,
            pltpu.SemaphoreType.DMA,
            pltpu.SemaphoreType.DMA,
            pltpu.SemaphoreType.DMA,
            pltpu.SemaphoreType.DMA,
            pltpu.SemaphoreType.DMA,
        ],
    )
    def k(hn_hbm, pk_hbm, z_hbm, out_hbm, pk_v, si_v, di_v,
          r0, r1, r2, acc_sh, g0, g1, g2, s0, s1, s2):
        cid = lax.axis_index("c")
        sid = lax.axis_index("s")
        gid = sid * _NC + cid
        pltpu.sync_copy(pk_hbm.at[gid], pk_v)

        @pl.when(sid < _ZT)
        def _():
            @pl.loop(0, _ZR // _ZB)
            def _(kk):
                pltpu.sync_copy(
                    z_hbm, acc_sh.at[pl.ds(sid * _ZR + kk * _ZB, _ZB)])

        plsc.subcore_barrier()

        rows = (r0, r1, r2)
        gsem = (g0, g1, g2)
        ssem = (s0, s1, s2)

        def unpack(b, j):
            @pl.loop(0, _C // 16)
            def _(u):
                v = pk_v[j, pl.ds(u * 16, 16)]
                si_v[b, pl.ds(u * 16, 16)] = v >> 14
                di_v[b, pl.ds(u * 16, 16)] = v & 16383

        def issue_gather(b, j):
            unpack(b, j)
            pltpu.make_async_copy(hn_hbm.at[si_v.at[b]], rows[b],
                                  gsem[b]).start()

        def wait_gather(b):
            pltpu.make_async_copy(hn_hbm.at[si_v.at[b]], rows[b],
                                  gsem[b]).wait()

        def scatter_async(b):
            pltpu.make_async_copy(rows[b], acc_sh.at[di_v.at[b]],
                                  ssem[b]).start(add=True)

        def wait_scatter(b):
            pltpu.make_async_copy(rows[b], acc_sh.at[di_v.at[b]],
                                  ssem[b]).wait()

        def visit(j, off, first=False):
            b = off % 3
            b2 = (off + 2) % 3
            wait_gather(b)
            scatter_async(b)
            if not first:
                wait_scatter(b2)
            issue_gather(b2, j + 2)

        issue_gather(0, 0)
        issue_gather(1, 1)
        visit(0, 0, first=True)
        visit(1, 1)
        visit(2, 2)

        @pl.loop(1, (_NCH - 2) // 3)
        def _(i):
            visit(3 * i, 0)
            visit(3 * i + 1, 1)
            visit(3 * i + 2, 2)

        wait_gather(0)
        scatter_async(0)
        wait_gather(1)
        scatter_async(1)
        wait_scatter(2)
        wait_scatter(0)
        wait_scatter(1)

        plsc.subcore_barrier()

        @pl.when(sid < _ZT)
        def _():
            @pl.loop(0, _ZR // _ZB)
            def _(kk):
                o = sid * _ZR + kk * _ZB
                pltpu.sync_copy(acc_sh.at[pl.ds(o, _ZB)],
                                out_hbm.at[cid, pl.ds(o, _ZB)])

    return k(hn, pk3, zrows)


def _phase_a_body(x_ref, w_ref, xw_ref, hn_ref):
    xw = jnp.dot(x_ref[...], w_ref[...], preferred_element_type=jnp.float32)
    xw_ref[...] = xw
    hn_ref[...] = jax.nn.sigmoid(xw[:, :_H]) * jnp.tanh(xw[:, 2 * _H:])


def _phase_a(x, w_all):
    return pl.pallas_call(
        _phase_a_body,
        grid=(_N // _R,),
        in_specs=[
            pl.BlockSpec((_R, _D), lambda i: (i, 0)),
            pl.BlockSpec((_D, 3 * _H), lambda i: (0, 0)),
        ],
        out_specs=[
            pl.BlockSpec((_R, 3 * _H), lambda i: (i, 0)),
            pl.BlockSpec((_R, _H), lambda i: (i, 0)),
        ],
        out_shape=[
            jax.ShapeDtypeStruct((_N, 3 * _H), jnp.float32),
            jax.ShapeDtypeStruct((_N, _H), jnp.float32),
        ],
    )(x, w_all)


def _gru_body(xw_ref, hn_ref, p0_ref, p1_ref, uz_ref, ur_ref, uh_ref, out_ref):
    f32 = jnp.float32
    m = p0_ref[0] + p1_ref[0]
    xw = xw_ref[...]
    hn = hn_ref[...]
    z = jax.nn.sigmoid(
        xw[:, :_H] + jnp.dot(m, uz_ref[...], preferred_element_type=f32))
    r = jax.nn.sigmoid(
        xw[:, _H:2 * _H]
        + jnp.dot(hn, ur_ref[...], preferred_element_type=f32))
    g = jnp.tanh(
        xw[:, 2 * _H:]
        + jnp.dot(r * m, uh_ref[...], preferred_element_type=f32))
    out_ref[...] = (1.0 - z) * m + z * g


def _gru(xw, hn, parts, u_z, u_r, u_h):
    nb = _N // _R
    return pl.pallas_call(
        _gru_body,
        grid=(nb,),
        in_specs=[
            pl.BlockSpec((_R, 3 * _H), lambda i: (i, 0)),
            pl.BlockSpec((_R, _H), lambda i: (i, 0)),
            pl.BlockSpec((1, _R, _H), lambda i: (0, i, 0)),
            pl.BlockSpec((1, _R, _H), lambda i: (1, i, 0)),
            pl.BlockSpec((_H, _H), lambda i: (0, 0)),
            pl.BlockSpec((_H, _H), lambda i: (0, 0)),
            pl.BlockSpec((_H, _H), lambda i: (0, 0)),
        ],
        out_specs=pl.BlockSpec((_R, _H), lambda i: (i, 0)),
        out_shape=jax.ShapeDtypeStruct((_N, _H), jnp.float32),
    )(xw, hn, parts, parts, u_z, u_r, u_h)


def _final_body(x_ref, p_ref, wot_ref, wob_ref, tgm_ref, tgv_ref, eps_ref,
                out_ref):
    f32 = jnp.float32
    m = p_ref[0] + p_ref[1]
    nh = jax.nn.relu(
        jnp.dot(x_ref[...], wot_ref[...], preferred_element_type=f32)
        + jnp.dot(m, wob_ref[...], preferred_element_type=f32))
    npb = _N // _B
    rows = lax.broadcasted_iota(jnp.int32, (_B, _N), 0)
    cols = lax.broadcasted_iota(jnp.int32, (_B, _N), 1)
    pool = jnp.where(cols // npb == rows, 1.0 / npb, 0.0)
    gv = jnp.dot(pool, nh, preferred_element_type=f32)
    mean = jnp.dot(gv, tgm_ref[...], preferred_element_type=f32)
    logv = -jnp.abs(jnp.dot(gv, tgv_ref[...], preferred_element_type=f32))
    out_ref[...] = mean + jnp.exp(logv * 0.5) * eps_ref[...]


def _final(x, parts, wo_t, wo_b, tgm, tgv, eps):
    return pl.pallas_call(
        _final_body,
        out_shape=jax.ShapeDtypeStruct((_B, _L), jnp.float32),
    )(x, parts, wo_t, wo_b, tgm, tgv, eps)


def kernel(x, edge_index, W_z, U_z, W_r, U_r, W_h, U_h, W_o,
           T_mean, T_var, G_mean, G_var, eps_t, eps_g):
    src = edge_index[0].astype(jnp.int32)
    dst = edge_index[1].astype(jnp.int32)
    pk3 = (src * 16384 + dst).reshape(_NT, _NCH, _C)
    zrows = jnp.zeros((_ZB, _H), jnp.float32)
    w_all = jnp.concatenate([W_z, W_r, W_h], axis=1)

    xw, hn = _phase_a(x, w_all)
    for _ in range(_DEPTH - 1):
        parts = _spmm(hn, pk3, zrows)
        hn = _gru(xw, hn, parts, U_z, U_r, U_h)
    parts = _spmm(hn, pk3, zrows)

    wo_t = W_o[:_D]
    wo_b = W_o[_D:]
    tgm = jnp.concatenate([T_mean, G_mean], axis=1)
    tgv = jnp.concatenate([T_var, G_var], axis=1)
    eps = jnp.concatenate([eps_t, eps_g], axis=1)
    return _final(x, parts, wo_t, wo_b, tgm, tgv, eps)
